# baseline (speedup 1.0000x reference)
import functools

import jax
import jax.numpy as jnp
from jax import lax
from jax.experimental import pallas as pl
from jax.experimental.pallas import tpu as pltpu
from jax.experimental.pallas import tpu_sc as plsc

F32 = jnp.float32

_NC = 2
_NS = 16
_NT = _NC * _NS
_L = 16

_NG = 4
_MI = 24
_MO = 24


def _round_up(x, m):
    return (x + m - 1) // m * m


@functools.cache
def _make_sc_layer(E, B, nfp, H):
    NV = B // _L
    NPT = nfp // _NT
    NGRP = NPT // _NG
    KI = _NG * _MI
    KO = _NG * _MO
    assert KI <= 128 and KO <= 128
    mesh = plsc.VectorSubcoreMesh(core_axis_name="c", subcore_axis_name="s")

    @functools.partial(
        pl.kernel,
        mesh=mesh,
        out_type=(),
        compiler_params=pltpu.CompilerParams(use_tc_tiling_on_sc=False),
        scratch_types=[
            pltpu.VMEM((KI,), jnp.int32),
            pltpu.VMEM((KI, B), F32),
            pltpu.VMEM((KI * H + 8,), F32),
            pltpu.VMEM((_NG * H + 8,), F32),
            pltpu.VMEM((KO * _L,), F32),
            pltpu.VMEM((KO,), jnp.int32),
            pltpu.VMEM((KO, B), F32),
        ],
    )
    def layer(x_hbm, a_hbm, b1_hbm, bw_hbm, ein_hbm, eout_hbm, y_hbm,
              idx_v, g_v, a_v, b1_v, bw_v, oi_v, o_v):
        tid = lax.axis_index("s") * _NC + lax.axis_index("c")
        node0 = tid * NPT

        @pl.loop(0, NGRP)
        def _blk(jb):
            s = node0 + jb * _NG
            si = s * _MI
            so = s * _MO
            pltpu.sync_copy(ein_hbm.at[pl.ds(si, KI)], idx_v)
            pltpu.sync_copy(a_hbm.at[pl.ds(si * H, KI * H + 8)], a_v)
            pltpu.sync_copy(b1_hbm.at[pl.ds(s * H, _NG * H + 8)], b1_v)
            pltpu.sync_copy(bw_hbm.at[pl.ds(so * _L, KO * _L)], bw_v)
            pltpu.sync_copy(eout_hbm.at[pl.ds(so, KO)], oi_v)
            pltpu.sync_copy(x_hbm.at[idx_v], g_v)

            @pl.loop(0, _NG)
            def _node(nn):
                kb = nn * _MI
                vb1 = b1_v[pl.ds(nn * H, _L)]
                acc = [[jnp.full((_L,), vb1[hh], F32)
                        for _ in range(NV)] for hh in range(H)]
                for i in range(_MI):
                    r = kb + i
                    g = [g_v[r, pl.ds(v * _L, _L)] for v in range(NV)]
                    wv = a_v[pl.ds(r * H, _L)]
                    for hh in range(H):
                        aa = wv[hh]
                        for v in range(NV):
                            acc[hh][v] = acc[hh][v] + g[v] * aa
                h = [[jnp.where(acc[hh][v] > 0.0,
                                acc[hh][v],
                                jnp.exp(jnp.minimum(acc[hh][v], 0.0)) - 1.0)
                      for v in range(NV)] for hh in range(H)]
                ob = nn * _MO
                for jj in range(_MO):
                    r = ob + jj
                    wv = bw_v[pl.ds(r * _L, _L)]
                    o = [jnp.full((_L,), wv[H], F32) for _ in range(NV)]
                    for hh in range(H):
                        w = wv[hh]
                        for v in range(NV):
                            o[v] = o[v] + h[hh][v] * w
                    for v in range(NV):
                        o_v[r, pl.ds(v * _L, _L)] = o[v]

            pltpu.sync_copy(o_v, y_hbm.at[oi_v])

    return layer


def _transpose_to_edge_major(x0):
    B, E = x0.shape
    CE = 640

    def body(x_ref, o_ref):
        o_ref[...] = x_ref[...].T

    return pl.pallas_call(
        body,
        grid=(E // CE,),
        in_specs=[pl.BlockSpec((B, CE), lambda i: (0, i))],
        out_specs=pl.BlockSpec((CE, B), lambda i: (i, 0)),
        out_shape=jax.ShapeDtypeStruct((E, B), F32),
    )(x0)


def _add_rows(y, xT):
    E, B = xT.shape
    CR = 2000

    def body(a_ref, b_ref, o_ref):
        o_ref[...] = a_ref[...] + b_ref[...]

    return pl.pallas_call(
        body,
        grid=(E // CR,),
        in_specs=[pl.BlockSpec((CR, B), lambda i: (i, 0)),
                  pl.BlockSpec((CR, B), lambda i: (i, 0))],
        out_specs=pl.BlockSpec((CR, B), lambda i: (i, 0)),
        out_shape=jax.ShapeDtypeStruct((E, B), F32),
    )(y, xT)


def _final_output(ysl, x0):
    B, E = x0.shape
    CE = 640

    def body(y_ref, x_ref, o_ref):
        o_ref[...] = y_ref[...].T + x_ref[...]

    return pl.pallas_call(
        body,
        grid=(E // CE,),
        in_specs=[pl.BlockSpec((CE, B), lambda i: (i, 0)),
                  pl.BlockSpec((B, CE), lambda i: (0, i))],
        out_specs=pl.BlockSpec((B, CE), lambda i: (0, i)),
        out_shape=jax.ShapeDtypeStruct((B, E), F32),
    )(ysl, x0)


def kernel(x0, W1, b1, W2, b2, in_pad, out_pad):
    B, E = x0.shape
    nf, H, max_in = W1.shape
    max_out = W2.shape[1]

    nfp = _round_up(nf, _NT * _NG)
    pad = nfp - nf
    pi = _MI - max_in
    po = _MO - max_out
    Af = jnp.pad(W1.transpose(0, 2, 1), ((0, pad), (0, pi), (0, 0)))
    Af = jnp.pad(Af.reshape(nfp * _MI * H), ((0, 8),))
    W2p = jnp.pad(W2, ((0, pad), (0, po), (0, 0)))
    b2p = jnp.pad(b2, ((0, pad), (0, po)))
    Bw = jnp.concatenate(
        [W2p, b2p[:, :, None],
         jnp.zeros((nfp, _MO, _L - H - 1), F32)], axis=-1)
    Bw = Bw.reshape(nfp * _MO * _L)
    b1f = jnp.pad(b1, ((0, pad), (0, 0))).reshape(nfp * H)
    b1f = jnp.pad(b1f, ((0, 8),))
    einf = jnp.pad(in_pad, ((0, pad), (0, pi))).reshape(-1)
    eoutf = jnp.pad(out_pad, ((0, pad), (0, po)),
                    constant_values=E).reshape(-1)

    YR = E + 8
    layer = _make_sc_layer(E, B, nfp, H)

    xT0 = _transpose_to_edge_major(x0)

    y1_ref = jax.new_ref(jnp.zeros((YR, B), F32))
    layer(xT0, Af, b1f, Bw, einf, eoutf, y1_ref)
    x1T = _add_rows(y1_ref[...][:E], xT0)

    y2_ref = jax.new_ref(jnp.zeros((YR, B), F32))
    layer(x1T, Af, b1f, Bw, einf, eoutf, y2_ref)
    return _final_output(y2_ref[...][:E], x0)
